# Initial kernel scaffold; baseline (speedup 1.0000x reference)
import jax
import jax.numpy as jnp
from jax.experimental import pallas as pl


def kernel(x, edge_index, W1_self, W1_neigh, b1, W2_self, W2_neigh, b2):
    raise NotImplementedError("write your pallas kernel here")


# speedup vs baseline: 5.1208x; 5.1208x over previous
import functools

import jax
import jax.numpy as jnp
from jax import lax
from jax.experimental import pallas as pl
from jax.experimental.pallas import tpu as pltpu
from jax.experimental.pallas import tpu_sc as plsc

N_NODES = 10000
N_EDGES = 320000
D = 128

NC = 2
NS = 16
NW = NC * NS
EPW = N_EDGES // NW
CH = 80
NCHUNK = EPW // CH
NPAD = 10240
RPT = NPAD // NS
DEGW = 16


def _fill2d(ref, nrows, ncols, val):
    def row(i, c):
        def col(k, c2):
            ref[i, pl.ds(k * 16, 16)] = jnp.full((16,), val, jnp.float32)
            return c2
        return lax.fori_loop(0, ncols // 16, col, c)
    lax.fori_loop(0, nrows, row, 0)


def _sc_deg(dst):
    mesh = plsc.VectorSubcoreMesh(core_axis_name="c", subcore_axis_name="s")

    @functools.partial(
        pl.kernel,
        out_type=jax.ShapeDtypeStruct((NW, NPAD), jnp.float32),
        mesh=mesh,
        scratch_types=[
            pltpu.VMEM((CH,), jnp.int32),
            pltpu.VMEM((NPAD,), jnp.float32),
        ],
        compiler_params=pltpu.CompilerParams(needs_layout_passes=False),
    )
    def k(dst_hbm, deg_hbm, didx, dega):
        c = lax.axis_index("c")
        s = lax.axis_index("s")
        wid = s * NC + c

        def z(j, carry):
            dega[pl.ds(j * 16, 16)] = jnp.zeros((16,), jnp.float32)
            return carry
        lax.fori_loop(0, NPAD // 16, z, 0)

        ones = jnp.ones((16,), jnp.float32)

        def chunk(j, carry):
            eoff = wid * EPW + j * CH
            pltpu.sync_copy(dst_hbm.at[pl.ds(eoff, CH)], didx)

            def sub(k2, c2):
                idx = didx[pl.ds(k2 * 16, 16)]
                plsc.addupdate_scatter(dega, [idx], ones)
                return c2
            return lax.fori_loop(0, CH // 16, sub, carry)
        lax.fori_loop(0, NCHUNK, chunk, 0)

        pltpu.sync_copy(dega, deg_hbm.at[wid])

    return k(dst)


def _tc_recip_deg(degp):
    RB = 1024

    def body(d_ref, o_ref):
        deg = jnp.sum(d_ref[...], axis=0)
        o_ref[...] = (1.0 / jnp.maximum(deg, 1.0)).reshape(RB, 1)

    return pl.pallas_call(
        body,
        grid=(NPAD // RB,),
        in_specs=[pl.BlockSpec((NW, RB), lambda i: (0, i))],
        out_specs=pl.BlockSpec((RB, 1), lambda i: (i, 0)),
        out_shape=jax.ShapeDtypeStruct((NPAD, 1), jnp.float32),
    )(degp)


def _sc_agg(h, src, dst):
    mesh = plsc.VectorSubcoreMesh(core_axis_name="c", subcore_axis_name="s")

    @functools.partial(
        pl.kernel,
        out_type=jax.ShapeDtypeStruct((NC, NPAD, D), jnp.float32),
        mesh=mesh,
        scratch_types=[
            pltpu.VMEM((CH,), jnp.int32),
            pltpu.VMEM((CH,), jnp.int32),
            pltpu.VMEM((CH, D), jnp.float32),
            pltpu.VMEM((32, D), jnp.float32),
            pltpu.VMEM_SHARED((NPAD, D), jnp.float32),
            pltpu.SemaphoreType.DMA,
        ],
    )
    def k(h_hbm, src_hbm, dst_hbm, out_hbm, sidx, didx, rows, zmain, acc, sem):
        c = lax.axis_index("c")
        s = lax.axis_index("s")
        wid = s * NC + c

        _fill2d(zmain, 32, D, 0.0)

        def z1(j, carry):
            pltpu.sync_copy(zmain, acc.at[pl.ds(s * RPT + j * 32, 32)])
            return carry
        lax.fori_loop(0, RPT // 32, z1, 0)

        plsc.subcore_barrier()

        def chunk(j, carry):
            eoff = wid * EPW + j * CH
            pltpu.sync_copy(src_hbm.at[pl.ds(eoff, CH)], sidx)
            pltpu.sync_copy(dst_hbm.at[pl.ds(eoff, CH)], didx)
            pltpu.async_copy(h_hbm.at[sidx], rows, sem).wait()
            pltpu.sync_copy(rows, acc.at[didx], add=True)
            return carry
        lax.fori_loop(0, NCHUNK, chunk, 0)

        plsc.subcore_barrier()

        pltpu.sync_copy(acc.at[pl.ds(s * RPT, RPT)],
                        out_hbm.at[c].at[pl.ds(s * RPT, RPT)])

    return k(h, src, dst)


def _tc_dense(h, p0, p1, recip, w_self, w_neigh, b, relu):
    R = 1000

    def body(h_ref, p0_ref, p1_ref, r_ref, ws_ref, wn_ref, b_ref, o_ref):
        neigh = (p0_ref[...] + p1_ref[...]) * r_ref[...]
        acc = jnp.dot(h_ref[...], ws_ref[...],
                      preferred_element_type=jnp.float32)
        acc = acc + jnp.dot(neigh, wn_ref[...],
                            preferred_element_type=jnp.float32)
        acc = acc + b_ref[...]
        if relu:
            acc = jnp.maximum(acc, 0.0)
        o_ref[...] = acc

    return pl.pallas_call(
        body,
        grid=(N_NODES // R,),
        in_specs=[
            pl.BlockSpec((R, D), lambda i: (i, 0)),
            pl.BlockSpec((R, D), lambda i: (i, 0)),
            pl.BlockSpec((R, D), lambda i: (i, 0)),
            pl.BlockSpec((R, 1), lambda i: (i, 0)),
            pl.BlockSpec((D, D), lambda i: (0, 0)),
            pl.BlockSpec((D, D), lambda i: (0, 0)),
            pl.BlockSpec((1, D), lambda i: (0, 0)),
        ],
        out_specs=pl.BlockSpec((R, D), lambda i: (i, 0)),
        out_shape=jax.ShapeDtypeStruct((N_NODES, D), jnp.float32),
    )(h, p0, p1, recip, w_self, w_neigh, b.reshape(1, D))


def kernel(x, edge_index, W1_self, W1_neigh, b1, W2_self, W2_neigh, b2):
    src = edge_index[0]
    dst = edge_index[1]

    degp = _sc_deg(dst)
    recip = _tc_recip_deg(degp)[:N_NODES]
    p1 = _sc_agg(x, src, dst)
    h1 = _tc_dense(x, p1[0, :N_NODES], p1[1, :N_NODES], recip,
                   W1_self, W1_neigh, b1, relu=True)
    p2 = _sc_agg(h1, src, dst)
    h2 = _tc_dense(h1, p2[0, :N_NODES], p2[1, :N_NODES], recip,
                   W2_self, W2_neigh, b2, relu=False)
    return h2


# speedup vs baseline: 9.6342x; 1.8814x over previous
import functools

import jax
import jax.numpy as jnp
from jax import lax
from jax.experimental import pallas as pl
from jax.experimental.pallas import tpu as pltpu
from jax.experimental.pallas import tpu_sc as plsc

N_NODES = 10000
N_EDGES = 320000
D = 128

NC = 2
NS = 16
NW = NC * NS
EPW = N_EDGES // NW
CH = 80
NCHUNK = EPW // CH
NPAD = 10240
RPT = NPAD // NS
DEGW = 16


def _fill2d(ref, nrows, ncols, val):
    def row(i, c):
        def col(k, c2):
            ref[i, pl.ds(k * 16, 16)] = jnp.full((16,), val, jnp.float32)
            return c2
        return lax.fori_loop(0, ncols // 16, col, c)
    lax.fori_loop(0, nrows, row, 0)


NBUF = 2


def _sc_deg(dstr):
    mesh = plsc.VectorSubcoreMesh(core_axis_name="c", subcore_axis_name="s")

    @functools.partial(
        pl.kernel,
        out_type=jax.ShapeDtypeStruct((NW, NPAD), jnp.float32),
        mesh=mesh,
        scratch_types=[
            pltpu.VMEM((EPW,), jnp.int32),
            pltpu.VMEM((NPAD,), jnp.float32),
        ],
        compiler_params=pltpu.CompilerParams(needs_layout_passes=False),
    )
    def k(dst_hbm, deg_hbm, didx, dega):
        c = lax.axis_index("c")
        s = lax.axis_index("s")
        wid = s * NC + c

        def z(j, carry):
            dega[pl.ds(j * 16, 16)] = jnp.zeros((16,), jnp.float32)
            return carry
        lax.fori_loop(0, NPAD // 16, z, 0)

        pltpu.sync_copy(dst_hbm.at[wid], didx)

        ones = jnp.ones((16,), jnp.float32)

        def sub(k2, c2):
            idx = didx[pl.ds(k2 * 16, 16)]
            plsc.addupdate_scatter(dega, [idx], ones)
            return c2
        lax.fori_loop(0, EPW // 16, sub, 0)

        pltpu.sync_copy(dega, deg_hbm.at[wid])

    return k(dstr)


def _sc_agg(h, srcr, dstr):
    mesh = plsc.VectorSubcoreMesh(core_axis_name="c", subcore_axis_name="s")

    @functools.partial(
        pl.kernel,
        out_type=jax.ShapeDtypeStruct((NC, NPAD, D), jnp.float32),
        mesh=mesh,
        scratch_types=[
            pltpu.VMEM((EPW,), jnp.int32),
            pltpu.VMEM((EPW,), jnp.int32),
            pltpu.VMEM((NBUF, CH, D), jnp.float32),
            pltpu.VMEM((8, D), jnp.float32),
            pltpu.VMEM_SHARED((NPAD, D), jnp.float32),
            pltpu.SemaphoreType.DMA,
            pltpu.SemaphoreType.DMA,
        ],
    )
    def body(h_hbm, src_hbm, dst_hbm, out_hbm,
             sidx, didx, rows, zmain, acc, gsem, ssem):
        c = lax.axis_index("c")
        s = lax.axis_index("s")
        wid = s * NC + c

        _fill2d(zmain, 8, D, 0.0)

        def z1(j, carry):
            pltpu.sync_copy(zmain, acc.at[pl.ds(s * RPT + j * 8, 8)])
            return carry
        lax.fori_loop(0, RPT // 8, z1, 0)

        pltpu.sync_copy(src_hbm.at[wid], sidx)
        pltpu.sync_copy(dst_hbm.at[wid], didx)

        plsc.subcore_barrier()

        def g_start(jj, b):
            pltpu.async_copy(
                h_hbm.at[sidx.at[pl.ds(jj * CH, CH)]], rows.at[b], gsem)

        def g_wait(b):
            pltpu.make_async_copy(
                h_hbm.at[sidx.at[pl.ds(0, CH)]], rows.at[b], gsem).wait()

        def s_start(jj, b):
            pltpu.async_copy(rows.at[b],
                             acc.at[didx.at[pl.ds(jj * CH, CH)]],
                             ssem, add=True)

        def s_drain(b):
            pltpu.make_async_copy(
                rows.at[b], acc.at[didx.at[pl.ds(0, CH)]], ssem).wait()

        g_start(jnp.int32(0), 0)
        g_start(jnp.int32(1), 1)

        g_wait(0)
        s_start(jnp.int32(0), 0)

        def duo(t, carry):
            for b2 in range(NBUF):
                jj = 1 + t * NBUF + b2
                b = (1 + b2) % NBUF
                g_wait(b)
                s_start(jj, b)
                s_drain(1 - b)
                nxt = jj + 1

                @pl.when(nxt < NCHUNK)
                def _():
                    g_start(nxt, 1 - b)
            return carry
        lax.fori_loop(0, (NCHUNK - 1) // NBUF, duo, 0)

        s_drain(0)

        plsc.subcore_barrier()

        pltpu.sync_copy(acc.at[pl.ds(s * RPT, RPT)],
                        out_hbm.at[c].at[pl.ds(s * RPT, RPT)])

    return body(h, srcr, dstr)


def _tc_recip_deg(degp):
    RB = 1024

    def body(d_ref, o_ref):
        deg = jnp.sum(d_ref[...], axis=0)
        o_ref[...] = (1.0 / jnp.maximum(deg, 1.0)).reshape(RB, 1)

    return pl.pallas_call(
        body,
        grid=(NPAD // RB,),
        in_specs=[pl.BlockSpec((NW, RB), lambda i: (0, i))],
        out_specs=pl.BlockSpec((RB, 1), lambda i: (i, 0)),
        out_shape=jax.ShapeDtypeStruct((NPAD, 1), jnp.float32),
    )(degp)


def _tc_recip_deg(degp):
    RB = 1024

    def body(d_ref, o_ref):
        deg = jnp.sum(d_ref[...], axis=0)
        o_ref[...] = (1.0 / jnp.maximum(deg, 1.0)).reshape(RB, 1)

    return pl.pallas_call(
        body,
        grid=(NPAD // RB,),
        in_specs=[pl.BlockSpec((NW, RB), lambda i: (0, i))],
        out_specs=pl.BlockSpec((RB, 1), lambda i: (i, 0)),
        out_shape=jax.ShapeDtypeStruct((NPAD, 1), jnp.float32),
    )(degp)


def _tc_dense(h, p0, p1, recip, w_self, w_neigh, b, relu):
    R = 1000

    def body(h_ref, p0_ref, p1_ref, r_ref, ws_ref, wn_ref, b_ref, o_ref):
        neigh = (p0_ref[...] + p1_ref[...]) * r_ref[...]
        acc = jnp.dot(h_ref[...], ws_ref[...],
                      preferred_element_type=jnp.float32)
        acc = acc + jnp.dot(neigh, wn_ref[...],
                            preferred_element_type=jnp.float32)
        acc = acc + b_ref[...]
        if relu:
            acc = jnp.maximum(acc, 0.0)
        o_ref[...] = acc

    return pl.pallas_call(
        body,
        grid=(N_NODES // R,),
        in_specs=[
            pl.BlockSpec((R, D), lambda i: (i, 0)),
            pl.BlockSpec((R, D), lambda i: (i, 0)),
            pl.BlockSpec((R, D), lambda i: (i, 0)),
            pl.BlockSpec((R, 1), lambda i: (i, 0)),
            pl.BlockSpec((D, D), lambda i: (0, 0)),
            pl.BlockSpec((D, D), lambda i: (0, 0)),
            pl.BlockSpec((1, D), lambda i: (0, 0)),
        ],
        out_specs=pl.BlockSpec((R, D), lambda i: (i, 0)),
        out_shape=jax.ShapeDtypeStruct((N_NODES, D), jnp.float32),
    )(h, p0, p1, recip, w_self, w_neigh, b.reshape(1, D))


def kernel(x, edge_index, W1_self, W1_neigh, b1, W2_self, W2_neigh, b2):
    src = edge_index[0]
    dst = edge_index[1]

    srcr = src.reshape(NW, EPW)
    dstr = dst.reshape(NW, EPW)

    degp = _sc_deg(dstr)
    recip = _tc_recip_deg(degp)[:N_NODES]
    p1 = _sc_agg(x, srcr, dstr)
    h1 = _tc_dense(x, p1[0, :N_NODES], p1[1, :N_NODES], recip,
                   W1_self, W1_neigh, b1, relu=True)
    p2 = _sc_agg(h1, srcr, dstr)
    h2 = _tc_dense(h1, p2[0, :N_NODES], p2[1, :N_NODES], recip,
                   W2_self, W2_neigh, b2, relu=False)
    return h2


# speedup vs baseline: 14.0715x; 1.4606x over previous
import functools

import jax
import jax.numpy as jnp
from jax import lax
from jax.experimental import pallas as pl
from jax.experimental.pallas import tpu as pltpu
from jax.experimental.pallas import tpu_sc as plsc

N_NODES = 10000
N_EDGES = 320000
D = 128

NC = 2
NS = 16
NW = NC * NS
EPW = N_EDGES // NW
CH = 80
NCHUNK = EPW // CH
NPAD = 10240
RPT = NPAD // NS
DEGW = 16


def _fill2d(ref, nrows, ncols, val):
    def row(i, c):
        def col(k, c2):
            ref[i, pl.ds(k * 16, 16)] = jnp.full((16,), val, jnp.float32)
            return c2
        return lax.fori_loop(0, ncols // 16, col, c)
    lax.fori_loop(0, nrows, row, 0)


NBUF = 4


def _sc_deg(dstr):
    mesh = plsc.VectorSubcoreMesh(core_axis_name="c", subcore_axis_name="s")

    @functools.partial(
        pl.kernel,
        out_type=jax.ShapeDtypeStruct((NW, NPAD), jnp.float32),
        mesh=mesh,
        scratch_types=[
            pltpu.VMEM((EPW,), jnp.int32),
            pltpu.VMEM((NPAD,), jnp.float32),
        ],
        compiler_params=pltpu.CompilerParams(needs_layout_passes=False),
    )
    def k(dst_hbm, deg_hbm, didx, dega):
        c = lax.axis_index("c")
        s = lax.axis_index("s")
        wid = s * NC + c

        def z(j, carry):
            dega[pl.ds(j * 16, 16)] = jnp.zeros((16,), jnp.float32)
            return carry
        lax.fori_loop(0, NPAD // 16, z, 0)

        pltpu.sync_copy(dst_hbm.at[wid], didx)

        ones = jnp.ones((16,), jnp.float32)

        def sub(k2, c2):
            idx = didx[pl.ds(k2 * 16, 16)]
            plsc.addupdate_scatter(dega, [idx], ones)
            return c2
        lax.fori_loop(0, EPW // 16, sub, 0)

        pltpu.sync_copy(dega, deg_hbm.at[wid])

    return k(dstr)


def _sc_agg(h, src, dst):
    mesh = plsc.VectorSubcoreMesh(core_axis_name="c", subcore_axis_name="s")

    @functools.partial(
        pl.kernel,
        out_type=jax.ShapeDtypeStruct((NC, NPAD, D), jnp.float32),
        mesh=mesh,
        scratch_types=[
            pltpu.VMEM((NBUF, CH), jnp.int32),
            pltpu.VMEM((NBUF, CH), jnp.int32),
            pltpu.VMEM((NBUF, CH, D), jnp.float32),
            pltpu.VMEM((8, D), jnp.float32),
            pltpu.VMEM_SHARED((NPAD, D), jnp.float32),
            pltpu.SemaphoreType.DMA,
            pltpu.SemaphoreType.DMA,
            pltpu.SemaphoreType.DMA,
            pltpu.SemaphoreType.DMA,
        ],
    )
    def body(h_hbm, src_hbm, dst_hbm, out_hbm,
             sidx, didx, rows, zmain, acc, s_isem, d_isem, gsem, ssem):
        c = lax.axis_index("c")
        s = lax.axis_index("s")
        wid = s * NC + c

        _fill2d(zmain, 8, D, 0.0)

        def z1(j, carry):
            pltpu.sync_copy(zmain, acc.at[pl.ds(s * RPT + j * 8, 8)])
            return carry
        lax.fori_loop(0, RPT // 8, z1, 0)

        plsc.subcore_barrier()

        ebase = wid * EPW

        def is_start(jj, b):
            pltpu.async_copy(src_hbm.at[pl.ds(ebase + jj * CH, CH)],
                             sidx.at[b], s_isem)

        def is_drain(b):
            pltpu.make_async_copy(src_hbm.at[pl.ds(0, CH)],
                                  sidx.at[b], s_isem).wait()

        def id_start(jj, b):
            pltpu.async_copy(dst_hbm.at[pl.ds(ebase + jj * CH, CH)],
                             didx.at[b], d_isem)

        def id_drain(b):
            pltpu.make_async_copy(dst_hbm.at[pl.ds(0, CH)],
                                  didx.at[b], d_isem).wait()

        def g_start(b):
            pltpu.async_copy(h_hbm.at[sidx.at[b]], rows.at[b], gsem)

        def g_wait(b):
            pltpu.make_async_copy(h_hbm.at[sidx.at[b]], rows.at[b],
                                  gsem).wait()

        def s_start(b):
            pltpu.async_copy(rows.at[b], acc.at[didx.at[b]], ssem, add=True)

        def s_drain(b):
            pltpu.make_async_copy(rows.at[b], acc.at[didx.at[b]],
                                  ssem).wait()

        for j in range(NBUF):
            is_start(jnp.int32(j), j)
            id_start(jnp.int32(j), j)
        for j in range(NBUF - 1):
            is_drain(j)
            g_start(j)
        g_wait(0)
        id_drain(0)
        s_start(0)
        is_drain(3)
        g_start(3)
        is_start(jnp.int32(NBUF), 0)

        def quad(t, carry):
            for b4 in range(NBUF):
                jj = 1 + t * NBUF + b4
                b = (1 + b4) % NBUF
                bp = (b - 1) % NBUF
                g_wait(b)
                id_drain(b)
                s_start(b)
                s_drain(bp)

                @pl.when(jj + 3 < NCHUNK)
                def _():
                    id_start(jj + 3, bp)
                    is_drain(bp)
                    g_start(bp)

                @pl.when(jj + 4 < NCHUNK)
                def _():
                    is_start(jj + 4, b)
            return carry
        lax.fori_loop(0, (NCHUNK - 1) // NBUF, quad, 0)

        s_drain(0)

        plsc.subcore_barrier()

        pltpu.sync_copy(acc.at[pl.ds(s * RPT, RPT)],
                        out_hbm.at[c].at[pl.ds(s * RPT, RPT)])

    return body(h, src, dst)


def _tc_recip_deg(degp):
    RB = 1024

    def body(d_ref, o_ref):
        deg = jnp.sum(d_ref[...], axis=0)
        o_ref[...] = (1.0 / jnp.maximum(deg, 1.0)).reshape(RB, 1)

    return pl.pallas_call(
        body,
        grid=(NPAD // RB,),
        in_specs=[pl.BlockSpec((NW, RB), lambda i: (0, i))],
        out_specs=pl.BlockSpec((RB, 1), lambda i: (i, 0)),
        out_shape=jax.ShapeDtypeStruct((NPAD, 1), jnp.float32),
    )(degp)


def _tc_recip_deg(degp):
    RB = 1024

    def body(d_ref, o_ref):
        deg = jnp.sum(d_ref[...], axis=0)
        o_ref[...] = (1.0 / jnp.maximum(deg, 1.0)).reshape(RB, 1)

    return pl.pallas_call(
        body,
        grid=(NPAD // RB,),
        in_specs=[pl.BlockSpec((NW, RB), lambda i: (0, i))],
        out_specs=pl.BlockSpec((RB, 1), lambda i: (i, 0)),
        out_shape=jax.ShapeDtypeStruct((NPAD, 1), jnp.float32),
    )(degp)


def _tc_dense(h, p0, p1, recip, w_self, w_neigh, b, relu):
    R = 1000

    def body(h_ref, p0_ref, p1_ref, r_ref, ws_ref, wn_ref, b_ref, o_ref):
        neigh = (p0_ref[...] + p1_ref[...]) * r_ref[...]
        acc = jnp.dot(h_ref[...], ws_ref[...],
                      preferred_element_type=jnp.float32)
        acc = acc + jnp.dot(neigh, wn_ref[...],
                            preferred_element_type=jnp.float32)
        acc = acc + b_ref[...]
        if relu:
            acc = jnp.maximum(acc, 0.0)
        o_ref[...] = acc

    return pl.pallas_call(
        body,
        grid=(N_NODES // R,),
        in_specs=[
            pl.BlockSpec((R, D), lambda i: (i, 0)),
            pl.BlockSpec((R, D), lambda i: (i, 0)),
            pl.BlockSpec((R, D), lambda i: (i, 0)),
            pl.BlockSpec((R, 1), lambda i: (i, 0)),
            pl.BlockSpec((D, D), lambda i: (0, 0)),
            pl.BlockSpec((D, D), lambda i: (0, 0)),
            pl.BlockSpec((1, D), lambda i: (0, 0)),
        ],
        out_specs=pl.BlockSpec((R, D), lambda i: (i, 0)),
        out_shape=jax.ShapeDtypeStruct((N_NODES, D), jnp.float32),
    )(h, p0, p1, recip, w_self, w_neigh, b.reshape(1, D))


def kernel(x, edge_index, W1_self, W1_neigh, b1, W2_self, W2_neigh, b2):
    src = edge_index[0]
    dst = edge_index[1]

    dstr = dst.reshape(NW, EPW)

    degp = _sc_deg(dstr)
    recip = _tc_recip_deg(degp)[:N_NODES]
    p1 = _sc_agg(x, src, dst)
    h1 = _tc_dense(x, p1[0, :N_NODES], p1[1, :N_NODES], recip,
                   W1_self, W1_neigh, b1, relu=True)
    p2 = _sc_agg(h1, src, dst)
    h2 = _tc_dense(h1, p2[0, :N_NODES], p2[1, :N_NODES], recip,
                   W2_self, W2_neigh, b2, relu=False)
    return h2


# speedup vs baseline: 14.6875x; 1.0438x over previous
import functools

import jax
import jax.numpy as jnp
from jax import lax
from jax.experimental import pallas as pl
from jax.experimental.pallas import tpu as pltpu
from jax.experimental.pallas import tpu_sc as plsc

N_NODES = 10000
N_EDGES = 320000
D = 128

NC = 2
NS = 16
NW = NC * NS
EPW = N_EDGES // NW
CH = 80
NCHUNK = EPW // CH
NPAD = 10240
RPT = NPAD // NS
DEGW = 16


def _fill2d(ref, nrows, ncols, val):
    def row(i, c):
        def col(k, c2):
            ref[i, pl.ds(k * 16, 16)] = jnp.full((16,), val, jnp.float32)
            return c2
        return lax.fori_loop(0, ncols // 16, col, c)
    lax.fori_loop(0, nrows, row, 0)


NBUF = 4


def _sc_deg(dstr):
    mesh = plsc.VectorSubcoreMesh(core_axis_name="c", subcore_axis_name="s")

    @functools.partial(
        pl.kernel,
        out_type=jax.ShapeDtypeStruct((NW, NPAD), jnp.float32),
        mesh=mesh,
        scratch_types=[
            pltpu.VMEM((EPW,), jnp.int32),
            pltpu.VMEM((NPAD,), jnp.float32),
        ],
        compiler_params=pltpu.CompilerParams(needs_layout_passes=False),
    )
    def k(dst_hbm, deg_hbm, didx, dega):
        c = lax.axis_index("c")
        s = lax.axis_index("s")
        wid = s * NC + c

        def z(j, carry):
            dega[pl.ds(j * 16, 16)] = jnp.zeros((16,), jnp.float32)
            return carry
        lax.fori_loop(0, NPAD // 16, z, 0)

        pltpu.sync_copy(dst_hbm.at[wid], didx)

        ones = jnp.ones((16,), jnp.float32)

        def sub(k2, c2):
            idx = didx[pl.ds(k2 * 16, 16)]
            plsc.addupdate_scatter(dega, [idx], ones)
            return c2
        lax.fori_loop(0, EPW // 16, sub, 0)

        pltpu.sync_copy(dega, deg_hbm.at[wid])

    return k(dstr)


def _sc_agg(h, src, dst):
    mesh = plsc.VectorSubcoreMesh(core_axis_name="c", subcore_axis_name="s")

    @functools.partial(
        pl.kernel,
        out_type=jax.ShapeDtypeStruct((NC, NPAD, D), jnp.float32),
        mesh=mesh,
        scratch_types=[
            pltpu.VMEM((NBUF, CH), jnp.int32),
            pltpu.VMEM((NBUF, CH), jnp.int32),
            pltpu.VMEM((NBUF, CH, D), jnp.float32),
            pltpu.VMEM((8, D), jnp.float32),
            pltpu.VMEM_SHARED((NPAD, D), jnp.float32),
            pltpu.SemaphoreType.DMA,
            pltpu.SemaphoreType.DMA,
            pltpu.SemaphoreType.DMA,
            pltpu.SemaphoreType.DMA,
        ],
    )
    def body(h_hbm, src_hbm, dst_hbm, out_hbm,
             sidx, didx, rows, zmain, acc, s_isem, d_isem, gsem, ssem):
        c = lax.axis_index("c")
        s = lax.axis_index("s")
        wid = s * NC + c

        _fill2d(zmain, 8, D, 0.0)

        def z1(j, carry):
            pltpu.sync_copy(zmain, acc.at[pl.ds(s * RPT + j * 8, 8)])
            return carry
        lax.fori_loop(0, RPT // 8, z1, 0)

        plsc.subcore_barrier()

        ebase = wid * EPW

        def is_start(jj, b):
            pltpu.async_copy(src_hbm.at[pl.ds(ebase + jj * CH, CH)],
                             sidx.at[b], s_isem)

        def is_drain(b):
            pltpu.make_async_copy(src_hbm.at[pl.ds(0, CH)],
                                  sidx.at[b], s_isem).wait()

        def id_start(jj, b):
            pltpu.async_copy(dst_hbm.at[pl.ds(ebase + jj * CH, CH)],
                             didx.at[b], d_isem)

        def id_drain(b):
            pltpu.make_async_copy(dst_hbm.at[pl.ds(0, CH)],
                                  didx.at[b], d_isem).wait()

        def g_start(b):
            pltpu.async_copy(h_hbm.at[sidx.at[b]], rows.at[b], gsem)

        def g_wait(b):
            pltpu.make_async_copy(h_hbm.at[sidx.at[b]], rows.at[b],
                                  gsem).wait()

        def s_start(b):
            pltpu.async_copy(rows.at[b], acc.at[didx.at[b]], ssem, add=True)

        def s_drain(b):
            pltpu.make_async_copy(rows.at[b], acc.at[didx.at[b]],
                                  ssem).wait()

        for j in range(NBUF):
            is_start(jnp.int32(j), j)
            id_start(jnp.int32(j), j)
        for j in range(NBUF - 1):
            is_drain(j)
            g_start(j)
        g_wait(0)
        id_drain(0)
        s_start(0)
        is_drain(3)
        g_start(3)
        is_start(jnp.int32(NBUF), 0)

        def quad(t, carry):
            for b4 in range(NBUF):
                jj = 1 + t * NBUF + b4
                b = (1 + b4) % NBUF
                bp = (b - 1) % NBUF
                g_wait(b)
                id_drain(b)
                s_start(b)
                s_drain(bp)

                @pl.when(jj + 3 < NCHUNK)
                def _():
                    id_start(jj + 3, bp)
                    is_drain(bp)
                    g_start(bp)

                @pl.when(jj + 4 < NCHUNK)
                def _():
                    is_start(jj + 4, b)
            return carry
        lax.fori_loop(0, (NCHUNK - 1) // NBUF, quad, 0)

        s_drain(0)

        plsc.subcore_barrier()

        pltpu.sync_copy(acc.at[pl.ds(s * RPT, RPT)],
                        out_hbm.at[c].at[pl.ds(s * RPT, RPT)])

    return body(h, src, dst)


def _tc_dense(h, p, degs, w_self, w_neigh, b, relu):
    R = 1024

    def body(h_ref, p0_ref, p1_ref, d_ref, ws_ref, wn_ref, b_ref, o_ref):
        deg = jnp.sum(d_ref[...], axis=0)
        recip = 1.0 / jnp.maximum(deg, 1.0)
        neigh = (p0_ref[0] + p1_ref[0]) * recip[:, None]
        acc = jnp.dot(h_ref[...], ws_ref[...],
                      preferred_element_type=jnp.float32)
        acc = acc + jnp.dot(neigh, wn_ref[...],
                            preferred_element_type=jnp.float32)
        acc = acc + b_ref[...]
        if relu:
            acc = jnp.maximum(acc, 0.0)
        o_ref[...] = acc

    return pl.pallas_call(
        body,
        grid=(NPAD // R,),
        in_specs=[
            pl.BlockSpec((R, D), lambda i: (i, 0)),
            pl.BlockSpec((1, R, D), lambda i: (0, i, 0)),
            pl.BlockSpec((1, R, D), lambda i: (1, i, 0)),
            pl.BlockSpec((NW, R), lambda i: (0, i)),
            pl.BlockSpec((D, D), lambda i: (0, 0)),
            pl.BlockSpec((D, D), lambda i: (0, 0)),
            pl.BlockSpec((1, D), lambda i: (0, 0)),
        ],
        out_specs=pl.BlockSpec((R, D), lambda i: (i, 0)),
        out_shape=jax.ShapeDtypeStruct((NPAD, D), jnp.float32),
    )(h, p, p, degs, w_self, w_neigh, b.reshape(1, D))


def kernel(x, edge_index, W1_self, W1_neigh, b1, W2_self, W2_neigh, b2):
    src = edge_index[0]
    dst = edge_index[1]

    dstr = dst.reshape(NW, EPW)
    x_pad = jnp.concatenate(
        [x, jnp.zeros((NPAD - N_NODES, D), jnp.float32)], axis=0)

    degp = _sc_deg(dstr)
    p1 = _sc_agg(x_pad, src, dst)
    h1 = _tc_dense(x_pad, p1, degp, W1_self, W1_neigh, b1, relu=True)
    p2 = _sc_agg(h1, src, dst)
    h2 = _tc_dense(h1, p2, degp, W2_self, W2_neigh, b2, relu=False)
    return h2[:N_NODES]
